# Initial kernel scaffold; baseline (speedup 1.0000x reference)
import jax
import jax.numpy as jnp
from jax.experimental import pallas as pl


def kernel(X, k):
    raise NotImplementedError("write your pallas kernel here")


# speedup vs baseline: 9.1529x; 9.1529x over previous
import functools

import jax
import jax.numpy as jnp
from jax.experimental import pallas as pl
from jax.experimental.pallas import tpu as pltpu

N = 4096
DIM = 256
K = 32
BLK = 256
INF = float("inf")


def _topk_kernel(x_blk_ref, x_full_ref, sq_ref, vals_ref, idx_ref, d_ref):
    i = pl.program_id(0)
    x_blk = x_blk_ref[...]
    x_full = x_full_ref[...]
    sq_full = sq_ref[...]

    s = jax.lax.dot_general(
        x_blk, x_full, (((1,), (1,)), ((), ())),
        preferred_element_type=jnp.float32,
    )
    sq_blk = jnp.sum(x_blk * x_blk, axis=1)
    d = sq_blk[:, None] + sq_full - 2.0 * s
    d = jnp.maximum(d, 0.0)

    col = jax.lax.broadcasted_iota(jnp.int32, (BLK, N), 1)
    row_g = i * BLK + jax.lax.broadcasted_iota(jnp.int32, (BLK, N), 0)
    d = jnp.where(col == row_g, INF, d)
    d_ref[...] = d

    def body(j, _):
        dd = d_ref[...]
        m = jnp.min(dd, axis=1)
        ii = jnp.where(dd == m[:, None], col, N)
        idx = jnp.min(ii, axis=1)
        vals_ref[pl.ds(j, 1), :] = m[None, :]
        idx_ref[pl.ds(j, 1), :] = idx[None, :]
        d_ref[...] = jnp.where(col == idx[:, None], INF, dd)
        return 0

    jax.lax.fori_loop(0, K, body, 0)


@jax.jit
def kernel(X, k):
    sq = jnp.sum(X * X, axis=1)[None, :]
    vals_t, idx_t = pl.pallas_call(
        _topk_kernel,
        grid=(N // BLK,),
        in_specs=[
            pl.BlockSpec((BLK, DIM), lambda i: (i, 0)),
            pl.BlockSpec((N, DIM), lambda i: (0, 0)),
            pl.BlockSpec((1, N), lambda i: (0, 0)),
        ],
        out_specs=[
            pl.BlockSpec((K, BLK), lambda i: (0, i)),
            pl.BlockSpec((K, BLK), lambda i: (0, i)),
        ],
        out_shape=[
            jax.ShapeDtypeStruct((K, N), jnp.float32),
            jax.ShapeDtypeStruct((K, N), jnp.int32),
        ],
        scratch_shapes=[pltpu.VMEM((BLK, N), jnp.float32)],
    )(X, X, sq)
    return vals_t.T, idx_t.T
